# speedup vs baseline: 11.3805x; 11.3805x over previous
import functools

import jax
import jax.numpy as jnp
from jax import lax
from jax.experimental import pallas as pl
from jax.experimental.pallas import tpu as pltpu
from jax.experimental.pallas import tpu_sc as plsc

N = 10000
E = 320000
D = 128
NC = 2
NS = 16
NW = NC * NS
NPAD = 10240
RPT = NPAD // NS
ZR = 128
C = 80
EPW = E // NW
NCHUNK = EPW // C

_mesh = plsc.VectorSubcoreMesh(core_axis_name="c", subcore_axis_name="s")

_f32 = jnp.float32


@functools.partial(
    pl.kernel,
    out_type=jax.ShapeDtypeStruct((NC, NPAD), _f32),
    mesh=_mesh,
    scratch_types=[
        pltpu.VMEM((C,), jnp.int32),
        pltpu.VMEM((C,), _f32),
        pltpu.VMEM((RPT,), _f32),
        pltpu.VMEM_SHARED((NPAD,), _f32),
        pltpu.SemaphoreType.DMA,
    ],
)
def _sc_deg(dst_hbm, out_hbm, dst_v, ones_v, zb_v, deg_sh, sem):
    c = lax.axis_index("c")
    s = lax.axis_index("s")
    wid = c * NS + s
    ones16 = jnp.ones((16,), _f32)
    zero16 = jnp.zeros((16,), _f32)
    for k in range(C // 16):
        ones_v[pl.ds(k * 16, 16)] = ones16

    def zb_body(i, carry):
        zb_v[pl.ds(i * 16, 16)] = zero16
        return carry

    lax.fori_loop(0, RPT // 16, zb_body, 0)
    pltpu.sync_copy(zb_v, deg_sh.at[pl.ds(s * RPT, RPT)])
    plsc.subcore_barrier()

    base = wid * EPW

    def body(j, carry):
        pltpu.sync_copy(dst_hbm.at[pl.ds(base + j * C, C)], dst_v)
        pltpu.sync_copy(ones_v, deg_sh.at[dst_v], add=True)
        return carry

    lax.fori_loop(0, NCHUNK, body, 0)
    plsc.subcore_barrier()
    pltpu.sync_copy(deg_sh.at[pl.ds(s * RPT, RPT)],
                    out_hbm.at[c, pl.ds(s * RPT, RPT)])


@functools.partial(
    pl.kernel,
    out_type=jax.ShapeDtypeStruct((NC, NPAD, D), _f32),
    mesh=_mesh,
    scratch_types=[
        pltpu.VMEM((C,), jnp.int32),
        pltpu.VMEM((C,), jnp.int32),
        pltpu.VMEM((C, D), _f32),
        pltpu.VMEM((ZR, D), _f32),
        pltpu.VMEM_SHARED((NPAD, D), _f32),
        pltpu.SemaphoreType.DMA,
    ],
)
def _sc_agg(hp_hbm, src_hbm, dst_hbm, out_hbm,
            src_v, dst_v, rows_v, zb_v, acc_sh, sem):
    c = lax.axis_index("c")
    s = lax.axis_index("s")
    wid = c * NS + s
    zero16 = jnp.zeros((16,), _f32)

    def zb_body(i, carry):
        for k in range(D // 16):
            zb_v[i, pl.ds(k * 16, 16)] = zero16
        return carry

    lax.fori_loop(0, ZR, zb_body, 0)
    for r in range(RPT // ZR):
        pltpu.sync_copy(zb_v, acc_sh.at[pl.ds(s * RPT + r * ZR, ZR)])
    plsc.subcore_barrier()

    base = wid * EPW

    def body(j, carry):
        off = base + j * C
        pltpu.sync_copy(src_hbm.at[pl.ds(off, C)], src_v)
        pltpu.sync_copy(dst_hbm.at[pl.ds(off, C)], dst_v)
        pltpu.async_copy(hp_hbm.at[src_v], rows_v, sem).wait()
        pltpu.sync_copy(rows_v, acc_sh.at[dst_v], add=True)
        return carry

    lax.fori_loop(0, NCHUNK, body, 0)
    plsc.subcore_barrier()
    pltpu.sync_copy(acc_sh.at[pl.ds(s * RPT, RPT)],
                    out_hbm.at[c, pl.ds(s * RPT, RPT)])


def _tc_first(x_ref, w_ref, dp_ref, hp_ref, dis_ref):
    deg = dp_ref[0, :N, :] + dp_ref[1, :N, :] + 1.0
    dis = lax.rsqrt(deg)
    dis_ref[...] = dis
    h = jnp.dot(x_ref[...], w_ref[...], preferred_element_type=_f32)
    hp_ref[...] = h * dis


def _tc_mid(ap_ref, hp_ref, dis_ref, g_ref, b_ref, bias_ref, w_ref, out_ref):
    dis = dis_ref[...]
    acc = ap_ref[0, :N, :] + ap_ref[1, :N, :]
    out = dis * (acc + hp_ref[...]) + bias_ref[...]
    mu = jnp.mean(out, axis=-1, keepdims=True)
    var = jnp.mean((out - mu) ** 2, axis=-1, keepdims=True)
    xln = (out - mu) * lax.rsqrt(var + 1e-5) * g_ref[...] + b_ref[...]
    x1 = jnp.maximum(xln, 0.0)
    out_ref[...] = jnp.dot(x1, w_ref[...], preferred_element_type=_f32) * dis


def _tc_last(ap_ref, hp_ref, dis_ref, g_ref, b_ref, bias_ref,
             a1_ref, ab1_ref, a2_ref, ab2_ref, x3_ref, gr_ref):
    dis = dis_ref[...]
    acc = ap_ref[0, :N, :] + ap_ref[1, :N, :]
    out = dis * (acc + hp_ref[...]) + bias_ref[...]
    mu = jnp.mean(out, axis=-1, keepdims=True)
    var = jnp.mean((out - mu) ** 2, axis=-1, keepdims=True)
    x3 = (out - mu) * lax.rsqrt(var + 1e-5) * g_ref[...] + b_ref[...]
    x3_ref[...] = x3
    t = jnp.tanh(jnp.dot(x3, a1_ref[...], preferred_element_type=_f32)
                 + ab1_ref[...])
    sc = jnp.dot(t, a2_ref[...], preferred_element_type=_f32) + ab2_ref[...]
    m = jnp.max(sc)
    e = jnp.exp(sc - m)
    z = jnp.sum(e)
    gr_ref[...] = jnp.sum(e * x3, axis=0, keepdims=True) / z


_first = pl.pallas_call(
    _tc_first,
    out_shape=(jax.ShapeDtypeStruct((N, D), _f32),
               jax.ShapeDtypeStruct((N, 1), _f32)),
)

_mid = pl.pallas_call(
    _tc_mid,
    out_shape=jax.ShapeDtypeStruct((N, D), _f32),
)

_last = pl.pallas_call(
    _tc_last,
    out_shape=(jax.ShapeDtypeStruct((N, D), _f32),
               jax.ShapeDtypeStruct((1, D), _f32)),
)


def kernel(batch_features, edge_index, W1, b1, W2, b2, W3, b3,
           ln1_g, ln1_b, ln2_g, ln2_b, ln3_g, ln3_b, A1, Ab1, A2, Ab2):
    src = edge_index[0]
    dst = edge_index[1]
    row = lambda v: v.reshape(1, -1)

    deg_parts = _sc_deg(dst)
    dp = deg_parts.reshape(NC, NPAD, 1)

    h1p, dis = _first(batch_features, W1, dp)
    acc1 = _sc_agg(h1p, src, dst)
    h2p = _mid(acc1, h1p, dis, row(ln1_g), row(ln1_b), row(b1), W2)
    acc2 = _sc_agg(h2p, src, dst)
    h3p = _mid(acc2, h2p, dis, row(ln2_g), row(ln2_b), row(b2), W3)
    acc3 = _sc_agg(h3p, src, dst)
    x3, graph_rep = _last(acc3, h3p, dis, row(ln3_g), row(ln3_b), row(b3),
                          A1, row(Ab1), A2, Ab2.reshape(1, 1))
    return (x3, graph_rep)


# speedup vs baseline: 29.8208x; 2.6203x over previous
import functools

import jax
import jax.numpy as jnp
from jax import lax
from jax.experimental import pallas as pl
from jax.experimental.pallas import tpu as pltpu
from jax.experimental.pallas import tpu_sc as plsc

N = 10000
E = 320000
D = 128
NC = 2
NS = 16
NW = NC * NS
NPAD = 10240
RPT = NPAD // NS
EPW = E // NW
CD = 80
NCHUNKD = EPW // CD
C = 40
NCHUNK = EPW // C
K = 5
IB = 50
NB = NCHUNK // IB

_mesh = plsc.VectorSubcoreMesh(core_axis_name="c", subcore_axis_name="s")

_f32 = jnp.float32


@functools.partial(
    pl.kernel,
    out_type=jax.ShapeDtypeStruct((NC, NPAD), _f32),
    mesh=_mesh,
    scratch_types=[
        pltpu.VMEM((NCHUNKD, CD), jnp.int32),
        pltpu.VMEM((CD,), _f32),
        pltpu.VMEM((RPT,), _f32),
        pltpu.VMEM_SHARED((NPAD,), _f32),
        pltpu.SemaphoreType.DMA,
    ],
)
def _sc_deg(dst_hbm, out_hbm, idx_dst, ones_v, zb_v, deg_sh, sem):
    c = lax.axis_index("c")
    s = lax.axis_index("s")
    wid = c * NS + s
    ones16 = jnp.ones((16,), _f32)
    zero16 = jnp.zeros((16,), _f32)
    for k in range(CD // 16):
        ones_v[pl.ds(k * 16, 16)] = ones16

    def zb_body(i, carry):
        zb_v[pl.ds(i * 16, 16)] = zero16
        return carry

    lax.fori_loop(0, RPT // 16, zb_body, 0)
    pltpu.sync_copy(zb_v, deg_sh.at[pl.ds(s * RPT, RPT)])
    pltpu.sync_copy(dst_hbm.at[wid], idx_dst)
    plsc.subcore_barrier()

    def body(j, carry):
        pltpu.sync_copy(ones_v, deg_sh.at[idx_dst.at[j]], add=True)
        return carry

    lax.fori_loop(0, NCHUNKD, body, 0)
    plsc.subcore_barrier()
    pltpu.sync_copy(deg_sh.at[pl.ds(s * RPT, RPT)],
                    out_hbm.at[c, pl.ds(s * RPT, RPT)])


@functools.partial(
    pl.kernel,
    out_type=jax.ShapeDtypeStruct((NC, NPAD, D), _f32),
    mesh=_mesh,
    scratch_types=[
        pltpu.VMEM((IB, C), jnp.int32),
        pltpu.VMEM((IB, C), jnp.int32),
        pltpu.VMEM((K, C, D), _f32),
        pltpu.VMEM_SHARED((NPAD, D), _f32),
        pltpu.SemaphoreType.DMA,
    ],
)
def _sc_agg(hp_hbm, src_hbm, dst_hbm, out_hbm,
            idx_src, idx_dst, rows_v, acc_sh, sem):
    c = lax.axis_index("c")
    s = lax.axis_index("s")
    wid = c * NS + s
    zero16 = jnp.zeros((16,), _f32)

    def zb_body(i, carry):
        for k in range(D // 16):
            rows_v[0, i, pl.ds(k * 16, 16)] = zero16
        return carry

    lax.fori_loop(0, C, zb_body, 0)
    for r in range(RPT // C):
        pltpu.sync_copy(rows_v.at[0], acc_sh.at[pl.ds(s * RPT + r * C, C)])
    plsc.subcore_barrier()

    def blk_body(blk, carry):
        pltpu.sync_copy(src_hbm.at[wid, blk], idx_src)
        pltpu.sync_copy(dst_hbm.at[wid, blk], idx_dst)
        for b in range(K):
            pltpu.async_copy(hp_hbm.at[idx_src.at[b]], rows_v.at[b], sem)

        def grp(g, carry2):
            jbase = g * K
            for b in range(K):
                j = jbase + b
                pltpu.make_async_copy(hp_hbm.at[idx_src.at[j]],
                                      rows_v.at[b], sem).wait()
                pltpu.sync_copy(rows_v.at[b], acc_sh.at[idx_dst.at[j]],
                                add=True)
                pltpu.async_copy(hp_hbm.at[idx_src.at[j + K]],
                                 rows_v.at[b], sem)
            return carry2

        lax.fori_loop(0, IB // K - 1, grp, 0)
        for b in range(K):
            j = IB - K + b
            pltpu.make_async_copy(hp_hbm.at[idx_src.at[j]],
                                  rows_v.at[b], sem).wait()
            pltpu.sync_copy(rows_v.at[b], acc_sh.at[idx_dst.at[j]], add=True)
        return carry

    lax.fori_loop(0, NB, blk_body, 0)
    plsc.subcore_barrier()
    pltpu.sync_copy(acc_sh.at[pl.ds(s * RPT, RPT)],
                    out_hbm.at[c, pl.ds(s * RPT, RPT)])


def _tc_first(x_ref, w_ref, dp_ref, hp_ref, dis_ref):
    deg = dp_ref[0, :N, :] + dp_ref[1, :N, :] + 1.0
    dis = lax.rsqrt(deg)
    dis_ref[...] = dis
    h = jnp.dot(x_ref[...], w_ref[...], preferred_element_type=_f32)
    hp_ref[...] = h * dis


def _tc_mid(ap_ref, hp_ref, dis_ref, g_ref, b_ref, bias_ref, w_ref, out_ref):
    dis = dis_ref[...]
    acc = ap_ref[0, :N, :] + ap_ref[1, :N, :]
    out = dis * (acc + hp_ref[...]) + bias_ref[...]
    mu = jnp.mean(out, axis=-1, keepdims=True)
    var = jnp.mean((out - mu) ** 2, axis=-1, keepdims=True)
    xln = (out - mu) * lax.rsqrt(var + 1e-5) * g_ref[...] + b_ref[...]
    x1 = jnp.maximum(xln, 0.0)
    out_ref[...] = jnp.dot(x1, w_ref[...], preferred_element_type=_f32) * dis


def _tc_last(ap_ref, hp_ref, dis_ref, g_ref, b_ref, bias_ref,
             a1_ref, ab1_ref, a2_ref, ab2_ref, x3_ref, gr_ref):
    dis = dis_ref[...]
    acc = ap_ref[0, :N, :] + ap_ref[1, :N, :]
    out = dis * (acc + hp_ref[...]) + bias_ref[...]
    mu = jnp.mean(out, axis=-1, keepdims=True)
    var = jnp.mean((out - mu) ** 2, axis=-1, keepdims=True)
    x3 = (out - mu) * lax.rsqrt(var + 1e-5) * g_ref[...] + b_ref[...]
    x3_ref[...] = x3
    t = jnp.tanh(jnp.dot(x3, a1_ref[...], preferred_element_type=_f32)
                 + ab1_ref[...])
    sc = jnp.dot(t, a2_ref[...], preferred_element_type=_f32) + ab2_ref[...]
    m = jnp.max(sc)
    e = jnp.exp(sc - m)
    z = jnp.sum(e)
    gr_ref[...] = jnp.sum(e * x3, axis=0, keepdims=True) / z


_first = pl.pallas_call(
    _tc_first,
    out_shape=(jax.ShapeDtypeStruct((N, D), _f32),
               jax.ShapeDtypeStruct((N, 1), _f32)),
)

_mid = pl.pallas_call(
    _tc_mid,
    out_shape=jax.ShapeDtypeStruct((N, D), _f32),
)

_last = pl.pallas_call(
    _tc_last,
    out_shape=(jax.ShapeDtypeStruct((N, D), _f32),
               jax.ShapeDtypeStruct((1, D), _f32)),
)


def kernel(batch_features, edge_index, W1, b1, W2, b2, W3, b3,
           ln1_g, ln1_b, ln2_g, ln2_b, ln3_g, ln3_b, A1, Ab1, A2, Ab2):
    src = edge_index[0].reshape(NW, NB, IB, C)
    dst = edge_index[1].reshape(NW, NB, IB, C)
    dst_deg = edge_index[1].reshape(NW, NCHUNKD, CD)
    row = lambda v: v.reshape(1, -1)

    deg_parts = _sc_deg(dst_deg)
    dp = deg_parts.reshape(NC, NPAD, 1)

    h1p, dis = _first(batch_features, W1, dp)
    acc1 = _sc_agg(h1p, src, dst)
    h2p = _mid(acc1, h1p, dis, row(ln1_g), row(ln1_b), row(b1), W2)
    acc2 = _sc_agg(h2p, src, dst)
    h3p = _mid(acc2, h2p, dis, row(ln2_g), row(ln2_b), row(b2), W3)
    acc3 = _sc_agg(h3p, src, dst)
    x3, graph_rep = _last(acc3, h3p, dis, row(ln3_g), row(ln3_b), row(b3),
                          A1, row(Ab1), A2, Ab2.reshape(1, 1))
    return (x3, graph_rep)


# speedup vs baseline: 30.5755x; 1.0253x over previous
import functools

import jax
import jax.numpy as jnp
from jax import lax
from jax.experimental import pallas as pl
from jax.experimental.pallas import tpu as pltpu
from jax.experimental.pallas import tpu_sc as plsc

N = 10000
E = 320000
D = 128
NC = 2
NS = 16
NW = NC * NS
NPAD = 10240
RPT = NPAD // NS
EPW = E // NW
CD = 80
NCHUNKD = EPW // CD
C = 40
NCHUNK = EPW // C
K = 5
IB = 50
NB = NCHUNK // IB

_mesh = plsc.VectorSubcoreMesh(core_axis_name="c", subcore_axis_name="s")

_f32 = jnp.float32


@functools.partial(
    pl.kernel,
    out_type=jax.ShapeDtypeStruct((NC, NPAD), _f32),
    mesh=_mesh,
    scratch_types=[
        pltpu.VMEM((NCHUNKD, CD), jnp.int32),
        pltpu.VMEM((CD,), _f32),
        pltpu.VMEM((RPT,), _f32),
        pltpu.VMEM_SHARED((NPAD,), _f32),
        pltpu.SemaphoreType.DMA,
    ],
)
def _sc_deg(dst_hbm, out_hbm, idx_dst, ones_v, zb_v, deg_sh, sem):
    c = lax.axis_index("c")
    s = lax.axis_index("s")
    wid = c * NS + s
    ones16 = jnp.ones((16,), _f32)
    zero16 = jnp.zeros((16,), _f32)
    for k in range(CD // 16):
        ones_v[pl.ds(k * 16, 16)] = ones16

    def zb_body(i, carry):
        zb_v[pl.ds(i * 16, 16)] = zero16
        return carry

    lax.fori_loop(0, RPT // 16, zb_body, 0)
    pltpu.sync_copy(zb_v, deg_sh.at[pl.ds(s * RPT, RPT)])
    pltpu.sync_copy(dst_hbm.at[wid], idx_dst)
    plsc.subcore_barrier()

    def body(j, carry):
        pltpu.async_copy(ones_v, deg_sh.at[idx_dst.at[j]], sem, add=True)
        return carry

    lax.fori_loop(0, NCHUNKD, body, 0)

    def drain(j, carry):
        pltpu.make_async_copy(ones_v, deg_sh.at[idx_dst.at[j]], sem).wait()
        return carry

    lax.fori_loop(0, NCHUNKD, drain, 0)
    plsc.subcore_barrier()
    pltpu.sync_copy(deg_sh.at[pl.ds(s * RPT, RPT)],
                    out_hbm.at[c, pl.ds(s * RPT, RPT)])


@functools.partial(
    pl.kernel,
    out_type=jax.ShapeDtypeStruct((NC, NPAD, D), _f32),
    mesh=_mesh,
    scratch_types=[
        pltpu.VMEM((IB, C), jnp.int32),
        pltpu.VMEM((IB, C), jnp.int32),
        pltpu.VMEM((K, C, D), _f32),
        pltpu.VMEM_SHARED((NPAD, D), _f32),
        pltpu.SemaphoreType.DMA,
        pltpu.SemaphoreType.DMA,
    ],
)
def _sc_agg(hp_hbm, src_hbm, dst_hbm, out_hbm,
            idx_src, idx_dst, rows_v, acc_sh, sem, ssem):
    c = lax.axis_index("c")
    s = lax.axis_index("s")
    wid = c * NS + s
    zero16 = jnp.zeros((16,), _f32)

    def zb_body(i, carry):
        for k in range(D // 16):
            rows_v[0, i, pl.ds(k * 16, 16)] = zero16
        return carry

    lax.fori_loop(0, C, zb_body, 0)
    for r in range(RPT // C):
        pltpu.sync_copy(rows_v.at[0], acc_sh.at[pl.ds(s * RPT + r * C, C)])
    plsc.subcore_barrier()

    def wait_g(j, b):
        pltpu.make_async_copy(hp_hbm.at[idx_src.at[j]],
                              rows_v.at[b], sem).wait()

    def fire_g(j, b):
        pltpu.async_copy(hp_hbm.at[idx_src.at[j]], rows_v.at[b], sem)

    def fire_s(j, b):
        pltpu.async_copy(rows_v.at[b], acc_sh.at[idx_dst.at[j]], ssem,
                         add=True)

    def wait_s(j, b):
        pltpu.make_async_copy(rows_v.at[b], acc_sh.at[idx_dst.at[j]],
                              ssem).wait()

    def blk_body(blk, carry):
        pltpu.sync_copy(src_hbm.at[wid, blk], idx_src)
        pltpu.sync_copy(dst_hbm.at[wid, blk], idx_dst)
        for b in range(K - 1):
            fire_g(b, b)
        wait_g(0, 0)
        fire_s(0, 0)
        fire_g(K - 1, K - 1)
        for j in range(1, K):
            b = j % K
            wait_g(j, b)
            fire_s(j, b)
            wait_s(j - 1, (j - 1) % K)
            fire_g(j + K - 1, (j - 1) % K)

        def grp(g, carry2):
            jbase = g * K
            for b in range(K):
                j = jbase + b
                wait_g(j, b)
                fire_s(j, b)
                wait_s(j - 1, (b - 1) % K)

                @pl.when(j + K - 1 < IB)
                def _():
                    fire_g(j + K - 1, (b - 1) % K)
            return carry2

        lax.fori_loop(1, IB // K, grp, 0)
        wait_s(IB - 1, (IB - 1) % K)
        return carry

    lax.fori_loop(0, NB, blk_body, 0)
    plsc.subcore_barrier()
    pltpu.sync_copy(acc_sh.at[pl.ds(s * RPT, RPT)],
                    out_hbm.at[c, pl.ds(s * RPT, RPT)])


def _tc_first(x_ref, w_ref, dp_ref, hp_ref, dis_ref):
    deg = dp_ref[0, :N, :] + dp_ref[1, :N, :] + 1.0
    dis = lax.rsqrt(deg)
    dis_ref[...] = dis
    h = jnp.dot(x_ref[...], w_ref[...], preferred_element_type=_f32)
    hp_ref[...] = h * dis


def _tc_mid(ap_ref, hp_ref, dis_ref, g_ref, b_ref, bias_ref, w_ref, out_ref):
    dis = dis_ref[...]
    acc = ap_ref[0, :N, :] + ap_ref[1, :N, :]
    out = dis * (acc + hp_ref[...]) + bias_ref[...]
    mu = jnp.mean(out, axis=-1, keepdims=True)
    var = jnp.mean((out - mu) ** 2, axis=-1, keepdims=True)
    xln = (out - mu) * lax.rsqrt(var + 1e-5) * g_ref[...] + b_ref[...]
    x1 = jnp.maximum(xln, 0.0)
    out_ref[...] = jnp.dot(x1, w_ref[...], preferred_element_type=_f32) * dis


def _tc_last(ap_ref, hp_ref, dis_ref, g_ref, b_ref, bias_ref,
             a1_ref, ab1_ref, a2_ref, ab2_ref, x3_ref, gr_ref):
    dis = dis_ref[...]
    acc = ap_ref[0, :N, :] + ap_ref[1, :N, :]
    out = dis * (acc + hp_ref[...]) + bias_ref[...]
    mu = jnp.mean(out, axis=-1, keepdims=True)
    var = jnp.mean((out - mu) ** 2, axis=-1, keepdims=True)
    x3 = (out - mu) * lax.rsqrt(var + 1e-5) * g_ref[...] + b_ref[...]
    x3_ref[...] = x3
    t = jnp.tanh(jnp.dot(x3, a1_ref[...], preferred_element_type=_f32)
                 + ab1_ref[...])
    sc = jnp.dot(t, a2_ref[...], preferred_element_type=_f32) + ab2_ref[...]
    m = jnp.max(sc)
    e = jnp.exp(sc - m)
    z = jnp.sum(e)
    gr_ref[...] = jnp.sum(e * x3, axis=0, keepdims=True) / z


_first = pl.pallas_call(
    _tc_first,
    out_shape=(jax.ShapeDtypeStruct((N, D), _f32),
               jax.ShapeDtypeStruct((N, 1), _f32)),
)

_mid = pl.pallas_call(
    _tc_mid,
    out_shape=jax.ShapeDtypeStruct((N, D), _f32),
)

_last = pl.pallas_call(
    _tc_last,
    out_shape=(jax.ShapeDtypeStruct((N, D), _f32),
               jax.ShapeDtypeStruct((1, D), _f32)),
)


def kernel(batch_features, edge_index, W1, b1, W2, b2, W3, b3,
           ln1_g, ln1_b, ln2_g, ln2_b, ln3_g, ln3_b, A1, Ab1, A2, Ab2):
    src = edge_index[0].reshape(NW, NB, IB, C)
    dst = edge_index[1].reshape(NW, NB, IB, C)
    dst_deg = edge_index[1].reshape(NW, NCHUNKD, CD)
    row = lambda v: v.reshape(1, -1)

    deg_parts = _sc_deg(dst_deg)
    dp = deg_parts.reshape(NC, NPAD, 1)

    h1p, dis = _first(batch_features, W1, dp)
    acc1 = _sc_agg(h1p, src, dst)
    h2p = _mid(acc1, h1p, dis, row(ln1_g), row(ln1_b), row(b1), W2)
    acc2 = _sc_agg(h2p, src, dst)
    h3p = _mid(acc2, h2p, dis, row(ln2_g), row(ln2_b), row(b2), W3)
    acc3 = _sc_agg(h3p, src, dst)
    x3, graph_rep = _last(acc3, h3p, dis, row(ln3_g), row(ln3_b), row(b3),
                          A1, row(Ab1), A2, Ab2.reshape(1, 1))
    return (x3, graph_rep)
